# Initial kernel scaffold; baseline (speedup 1.0000x reference)
import jax
import jax.numpy as jnp
from jax.experimental import pallas as pl


def kernel(beta, embed, slice_id, is_cp):
    raise NotImplementedError("write your pallas kernel here")


# speedup vs baseline: 7.9253x; 7.9253x over previous
import functools

import jax
import jax.numpy as jnp
from jax import lax
from jax.experimental import pallas as pl
from jax.experimental.pallas import tpu as pltpu

ATTR_W = 1.0
REP_W = 1.5
TAU = 0.7
SUPP_W = 0.5
CLAMP = 20.0
NUM_SLICES = 128

_B, _N, _D, _S = 4, 8192, 16, 128
_TR = 256


def _loss_kernel(beta_ref, emb_ref, sid_row_ref, cp_row_ref, sid_col_ref,
                 cp_col_ref, tot_ref, beta_out_ref, attr_out_ref,
                 repl_out_ref, acc_ref, embs_ref, njc_ref):
    b = pl.program_id(0)
    nb = pl.num_programs(0)

    @pl.when(b == 0)
    def _init():
        for i in range(8):
            acc_ref[i] = 0.0

    sid = sid_row_ref[0]
    cp = (cp_row_ref[0] == 1)
    bta = beta_ref[0]
    emb_raw = emb_ref[0]
    sid_c = sid_col_ref[0]
    cp_c = (cp_col_ref[0] == 1)

    b2 = jnp.where(jnp.isfinite(bta), bta, 0.0)
    b2 = jnp.clip(b2, -CLAMP, CLAMP)
    embc = jnp.where(jnp.isfinite(emb_raw), emb_raw, 0.0)

    e = jnp.exp(b2 / TAU - CLAMP / TAU)
    cpf = cp.astype(jnp.float32)

    iota_s = lax.broadcasted_iota(jnp.int32, (_S, _N), 0)
    member = (sid == iota_s)
    memf = member.astype(jnp.float32)

    ones_row = jnp.ones((1, _N), jnp.float32)
    vals = jnp.concatenate([ones_row, cpf, e, e * cpf], axis=0)
    sums = lax.dot_general(memf, vals, (((1,), (1,)), ((), ())),
                           preferred_element_type=jnp.float32)
    counts = sums[:, 0:1]
    cp_counts = sums[:, 1:2]
    z_raw = sums[:, 2:3]
    cpz = sums[:, 3:4]
    z = jnp.where(counts > 0, z_raw, 1.0)

    p_cp = cpz / z
    slice_ce = -jnp.log(p_cp + 1e-9)
    noncp_sum = (z_raw - cpz) / z
    noncp_n = counts - cp_counts
    supp = jnp.where(noncp_n > 0, noncp_sum / jnp.maximum(noncp_n, 1.0), 0.0)
    sel = cp_counts == 1.0
    slice_count = jnp.sum(sel.astype(jnp.float32))
    beta_loss = jnp.sum(jnp.where(sel, slice_ce + SUPP_W * supp, 0.0))
    beta_loss = beta_loss / jnp.maximum(slice_count, 1.0)

    iota_n = lax.broadcasted_iota(jnp.int32, (_S, _N), 1)
    cpmem = member & cp
    min_idx = jnp.min(jnp.where(cpmem, iota_n, _N), axis=1, keepdims=True)
    onehot = (iota_n == min_idx).astype(jnp.float32)
    cp_vec = lax.dot_general(onehot, embc, (((1,), (0,)), ((), ())),
                             preferred_element_type=jnp.float32)

    emb2 = embc * embc
    nj = lax.dot_general(jnp.ones((1, _D), jnp.float32), emb2,
                         (((1,), (1,)), ((), ())),
                         preferred_element_type=jnp.float32)
    ns = jnp.sum(cp_vec * cp_vec, axis=1, keepdims=True)
    cross = lax.dot_general(cp_vec, embc, (((1,), (1,)), ((), ())),
                            preferred_element_type=jnp.float32)
    d2 = jnp.maximum(nj + ns - 2.0 * cross, 0.0)
    d2 = jnp.minimum(d2, 50.0)
    d2sum = jnp.sum(memf * d2, axis=1, keepdims=True)
    d2_mean = d2sum / jnp.maximum(counts, 1.0)
    att_sel = cp_counts > 0
    att_count = jnp.sum(att_sel.astype(jnp.float32))
    attraction = jnp.sum(jnp.where(att_sel, d2_mean, 0.0))
    attraction = jnp.where(att_count > 0,
                           ATTR_W * attraction / jnp.maximum(att_count, 1.0),
                           0.0)

    k_total = jnp.sum(cp_counts)
    rep_den = k_total * k_total - jnp.sum(cp_counts * cp_counts)

    embs_ref[...] = embc
    njc_ref[...] = jnp.sum(emb2, axis=1, keepdims=True)

    def _rep_tile(t, acc):
        r0 = t * _TR
        emb_r = embs_ref[pl.ds(r0, _TR), :]
        gram = lax.dot_general(emb_r, embc, (((1,), (1,)), ((), ())),
                               preferred_element_type=jnp.float32)
        d2p = njc_ref[pl.ds(r0, _TR), :] + nj - 2.0 * gram
        d2p = jnp.minimum(jnp.maximum(d2p, 0.0), 50.0)
        pmask = ((cp_col_ref[0, pl.ds(r0, _TR), :] == 1) & cp
                 & (sid_col_ref[0, pl.ds(r0, _TR), :] != sid))
        return acc + jnp.sum(jnp.where(pmask, jnp.exp(-d2p), 0.0))

    rep_num = lax.fori_loop(0, _N // _TR, _rep_tile, 0.0)
    repulsion = jnp.where(rep_den > 0,
                          rep_num / jnp.maximum(rep_den, 1.0) * REP_W, 0.0)

    cp_total = jnp.sum(cpf)
    vf = ((cp_total > 0) & (slice_count > 0)).astype(jnp.float32)

    acc_ref[0] += vf * (beta_loss + attraction + repulsion)
    acc_ref[1] += vf * beta_loss
    acc_ref[2] += vf * attraction
    acc_ref[3] += vf * repulsion
    acc_ref[4] += vf

    @pl.when(b == nb - 1)
    def _fin():
        valid = acc_ref[4]
        denom = jnp.maximum(valid, 1.0)
        scale = jnp.where(valid > 0, 1.0 / denom, 0.0)
        one = jnp.ones((1, 1), jnp.float32)
        tot_ref[...] = one * (acc_ref[0] * scale)
        beta_out_ref[...] = one * (acc_ref[1] * scale)
        attr_out_ref[...] = one * (acc_ref[2] * scale)
        repl_out_ref[...] = one * (acc_ref[3] * scale)


def kernel(beta, embed, slice_id, is_cp):
    B, N, D = embed.shape
    beta_row = beta.reshape(B, 1, N)
    sid_row = slice_id.reshape(B, 1, N)
    cp_row = is_cp.reshape(B, 1, N)
    sid_col = slice_id.reshape(B, N, 1)
    cp_col = is_cp.reshape(B, N, 1)

    out_shape = [jax.ShapeDtypeStruct((1, 1), jnp.float32)] * 4
    scalar_spec = pl.BlockSpec((1, 1), lambda b: (0, 0))
    outs = pl.pallas_call(
        _loss_kernel,
        grid=(B,),
        in_specs=[
            pl.BlockSpec((1, 1, N), lambda b: (b, 0, 0)),
            pl.BlockSpec((1, N, D), lambda b: (b, 0, 0)),
            pl.BlockSpec((1, 1, N), lambda b: (b, 0, 0)),
            pl.BlockSpec((1, 1, N), lambda b: (b, 0, 0)),
            pl.BlockSpec((1, N, 1), lambda b: (b, 0, 0)),
            pl.BlockSpec((1, N, 1), lambda b: (b, 0, 0)),
        ],
        out_specs=[scalar_spec] * 4,
        out_shape=out_shape,
        scratch_shapes=[pltpu.SMEM((8,), jnp.float32),
                        pltpu.VMEM((N, D), jnp.float32),
                        pltpu.VMEM((N, 1), jnp.float32)],
    )(beta_row, embed, sid_row, cp_row, sid_col, cp_col)
    return tuple(o.reshape(()) for o in outs)


# speedup vs baseline: 47.3578x; 5.9755x over previous
import functools

import jax
import jax.numpy as jnp
from jax import lax
from jax.experimental import pallas as pl
from jax.experimental.pallas import tpu as pltpu
from jax.experimental.pallas import tpu_sc as plsc

ATTR_W = 1.0
REP_W = 1.5
TAU = 0.7
SUPP_W = 0.5
CLAMP = 20.0

_B, _N, _D, _S = 4, 8192, 16, 128
_NP = _N + 256
_TRP = 256
_CHUNK = _N // 8
_NV = _CHUNK // 16


def _sc_compact_kernel(cp_hbm, sid_hbm, emb_hbm, ce_hbm, csid_hbm,
                       cp_v, sid_v, lidx, lsid, cnt_v, tbl_sh, tbl_v,
                       srcb, dstb, rows, sidrows, sem):
    c = lax.axis_index("c")
    s = lax.axis_index("s")
    batch = c * 2 + s // 8
    rank = s % 8
    base = batch * _N + rank * _CHUNK
    lane = lax.iota(jnp.int32, 16)

    pltpu.async_copy(cp_hbm.at[pl.ds(base, _CHUNK)], cp_v, sem).wait()
    pltpu.async_copy(sid_hbm.at[pl.ds(base, _CHUNK)], sid_v, sem).wait()

    def body(i, wcount):
        o = i * 16
        cp16 = cp_v[pl.ds(o, 16)]
        sid16 = sid_v[pl.ds(o, 16)]
        m = cp16 == 1
        mi = jnp.where(m, 1, 0).astype(jnp.int32)
        cum = plsc.cumsum(mi)
        pos = wcount + cum - mi
        zeros = jnp.zeros((16,), jnp.int32)
        lidx[pl.ds(o, 16)] = zeros
        lsid[pl.ds(o, 16)] = zeros
        gidx = rank * _CHUNK + o + lane
        plsc.store_scatter(lidx, [pos], gidx, mask=m)
        plsc.store_scatter(lsid, [pos], sid16, mask=m)
        return wcount + jnp.sum(mi)

    wcount = lax.fori_loop(0, _NV, body, jnp.int32(0))

    cnt_v[...] = jnp.broadcast_to(wcount, (16,))
    pltpu.sync_copy(cnt_v, tbl_sh.at[s])
    plsc.subcore_barrier()
    pltpu.sync_copy(tbl_sh, tbl_v)
    g0 = (s // 8) * 8
    cnts = plsc.load_gather(tbl_v, [g0 + (lane & 7), jnp.zeros((16,), jnp.int32)])
    grp = jnp.where(lane < 8, cnts, 0)
    my_off = jnp.sum(jnp.where(lane < rank, grp, 0))

    dst_base = batch * _NP + my_off
    trash = batch * _NP + _N
    for ci in range(8):
        c0 = ci * 128

        @pl.when(wcount > c0)
        def _chunk():
            for v in range(8):
                o = c0 + v * 16
                li = lidx[pl.ds(o, 16)]
                ls = lsid[pl.ds(o, 16)]
                valid = (o + lane) < wcount
                srcb[pl.ds(v * 16, 16)] = batch * _N + li
                dstb[pl.ds(v * 16, 16)] = jnp.where(
                    valid, dst_base + o + lane, trash)
                plsc.store_scatter(
                    sidrows, [v * 16 + lane, jnp.zeros((16,), jnp.int32)], ls)
            pltpu.async_copy(emb_hbm.at[srcb], rows, sem).wait()
            pltpu.async_copy(rows, ce_hbm.at[dstb], sem).wait()
            pltpu.async_copy(sidrows, csid_hbm.at[dstb], sem).wait()


_sc_out_type = (jax.ShapeDtypeStruct((_B * _NP, _D), jnp.float32),
                jax.ShapeDtypeStruct((_B * _NP, 16), jnp.int32))
_sc_scratch = [
    pltpu.VMEM((_CHUNK,), jnp.int32),
    pltpu.VMEM((_CHUNK,), jnp.int32),
    pltpu.VMEM((_CHUNK,), jnp.int32),
    pltpu.VMEM((_CHUNK,), jnp.int32),
    pltpu.VMEM((16,), jnp.int32),
    pltpu.VMEM_SHARED((16, 16), jnp.int32),
    pltpu.VMEM((16, 16), jnp.int32),
    pltpu.VMEM((128,), jnp.int32),
    pltpu.VMEM((128,), jnp.int32),
    pltpu.VMEM((128, _D), jnp.float32),
    pltpu.VMEM((128, 16), jnp.int32),
    pltpu.SemaphoreType.DMA,
]


def _sc_compact(cp_flat, sid_flat, emb_flat):
    mesh = plsc.VectorSubcoreMesh(core_axis_name="c", subcore_axis_name="s")
    f = pl.kernel(_sc_compact_kernel, out_type=_sc_out_type, mesh=mesh,
                  scratch_types=_sc_scratch,
                  compiler_params=pltpu.CompilerParams(
                      needs_layout_passes=False,
                      use_tc_tiling_on_sc=False))
    return f(cp_flat, sid_flat, emb_flat)


def _loss_kernel(beta_ref, emb_ref, sid_row_ref, cp_row_ref, ce_ref,
                 csid_row_ref, csid_col_ref, tot_ref, beta_out_ref,
                 attr_out_ref, repl_out_ref, acc_ref):
    b = pl.program_id(0)
    nb = pl.num_programs(0)

    @pl.when(b == 0)
    def _init():
        for i in range(8):
            acc_ref[i] = 0.0

    sid = sid_row_ref[0]
    cp = (cp_row_ref[0] == 1)
    bta = beta_ref[0]
    emb_raw = emb_ref[0]

    b2 = jnp.where(jnp.isfinite(bta), bta, 0.0)
    b2 = jnp.clip(b2, -CLAMP, CLAMP)
    embc = jnp.where(jnp.isfinite(emb_raw), emb_raw, 0.0)

    e = jnp.exp(b2 / TAU - CLAMP / TAU)
    cpf = cp.astype(jnp.float32)

    iota_s = lax.broadcasted_iota(jnp.int32, (_S, _N), 0)
    member = (sid == iota_s)
    memf = member.astype(jnp.float32)

    ones_row = jnp.ones((1, _N), jnp.float32)
    vals = jnp.concatenate([ones_row, cpf, e, e * cpf], axis=0)
    sums = lax.dot_general(memf, vals, (((1,), (1,)), ((), ())),
                           preferred_element_type=jnp.float32)
    counts = sums[:, 0:1]
    cp_counts = sums[:, 1:2]
    z_raw = sums[:, 2:3]
    cpz = sums[:, 3:4]
    z = jnp.where(counts > 0, z_raw, 1.0)

    p_cp = cpz / z
    slice_ce = -jnp.log(p_cp + 1e-9)
    noncp_sum = (z_raw - cpz) / z
    noncp_n = counts - cp_counts
    supp = jnp.where(noncp_n > 0, noncp_sum / jnp.maximum(noncp_n, 1.0), 0.0)
    sel = cp_counts == 1.0
    slice_count = jnp.sum(sel.astype(jnp.float32))
    beta_loss = jnp.sum(jnp.where(sel, slice_ce + SUPP_W * supp, 0.0))
    beta_loss = beta_loss / jnp.maximum(slice_count, 1.0)

    iota_n = lax.broadcasted_iota(jnp.int32, (_S, _N), 1)
    cpmem = member & cp
    min_idx = jnp.min(jnp.where(cpmem, iota_n, _N), axis=1, keepdims=True)
    onehot = (iota_n == min_idx).astype(jnp.float32)
    cp_vec = lax.dot_general(onehot, embc, (((1,), (0,)), ((), ())),
                             preferred_element_type=jnp.float32)

    emb2 = embc * embc
    nj = lax.dot_general(jnp.ones((1, _D), jnp.float32), emb2,
                         (((1,), (1,)), ((), ())),
                         preferred_element_type=jnp.float32)
    ns = jnp.sum(cp_vec * cp_vec, axis=1, keepdims=True)
    cross = lax.dot_general(cp_vec, embc, (((1,), (1,)), ((), ())),
                            preferred_element_type=jnp.float32)
    d2 = jnp.maximum(nj + ns - 2.0 * cross, 0.0)
    d2 = jnp.minimum(d2, 50.0)
    d2sum = jnp.sum(memf * d2, axis=1, keepdims=True)
    d2_mean = d2sum / jnp.maximum(counts, 1.0)
    att_sel = cp_counts > 0
    att_count = jnp.sum(att_sel.astype(jnp.float32))
    attraction = jnp.sum(jnp.where(att_sel, d2_mean, 0.0))
    attraction = jnp.where(att_count > 0,
                           ATTR_W * attraction / jnp.maximum(att_count, 1.0),
                           0.0)

    k_total = jnp.sum(cp_counts)
    rep_den = k_total * k_total - jnp.sum(cp_counts * cp_counts)
    ki = k_total.astype(jnp.int32)
    ntiles = (ki + _TRP - 1) // _TRP
    ones_d = jnp.ones((1, _D), jnp.float32)

    def _rep_tile(t, acc):
        tr = t // ntiles
        tc = t % ntiles
        r0 = tr * _TRP
        c0 = tc * _TRP
        cer_raw = ce_ref[0, pl.ds(r0, _TRP), :]
        cec_raw = ce_ref[0, pl.ds(c0, _TRP), :]
        ce_r = jnp.where(jnp.isfinite(cer_raw), cer_raw, 0.0)
        ce_c = jnp.where(jnp.isfinite(cec_raw), cec_raw, 0.0)
        gram = lax.dot_general(ce_r, ce_c, (((1,), (1,)), ((), ())),
                               preferred_element_type=jnp.float32)
        nr = jnp.sum(ce_r * ce_r, axis=1, keepdims=True)
        nc = lax.dot_general(ones_d, ce_c * ce_c, (((1,), (1,)), ((), ())),
                             preferred_element_type=jnp.float32)
        d2p = jnp.minimum(jnp.maximum(nr + nc - 2.0 * gram, 0.0), 50.0)
        sr = csid_col_ref[0, pl.ds(r0, _TRP), :]
        sc = csid_row_ref[0, :, pl.ds(c0, _TRP)]
        ir = lax.broadcasted_iota(jnp.int32, (_TRP, 1), 0) + r0
        ic = lax.broadcasted_iota(jnp.int32, (1, _TRP), 1) + c0
        pmask = (ir < ki) & (ic < ki) & (sr != sc)
        return acc + jnp.sum(jnp.where(pmask, jnp.exp(-d2p), 0.0))

    rep_num = lax.fori_loop(0, ntiles * ntiles, _rep_tile, 0.0)
    repulsion = jnp.where(rep_den > 0,
                          rep_num / jnp.maximum(rep_den, 1.0) * REP_W, 0.0)

    cp_total = jnp.sum(cpf)
    vf = ((cp_total > 0) & (slice_count > 0)).astype(jnp.float32)

    acc_ref[0] += vf * (beta_loss + attraction + repulsion)
    acc_ref[1] += vf * beta_loss
    acc_ref[2] += vf * attraction
    acc_ref[3] += vf * repulsion
    acc_ref[4] += vf

    @pl.when(b == nb - 1)
    def _fin():
        valid = acc_ref[4]
        denom = jnp.maximum(valid, 1.0)
        scale = jnp.where(valid > 0, 1.0 / denom, 0.0)
        one = jnp.ones((1, 1), jnp.float32)
        tot_ref[...] = one * (acc_ref[0] * scale)
        beta_out_ref[...] = one * (acc_ref[1] * scale)
        attr_out_ref[...] = one * (acc_ref[2] * scale)
        repl_out_ref[...] = one * (acc_ref[3] * scale)


def kernel(beta, embed, slice_id, is_cp):
    B, N, D = embed.shape
    NP = _NP
    ce2, csid2 = _sc_compact(is_cp.reshape(B * N),
                             slice_id.reshape(B * N),
                             embed.reshape(B * N, D))
    ce = ce2.reshape(B, NP, D)
    csid = csid2[:, 0]
    csid_row = csid.reshape(B, 1, NP)
    csid_col = csid.reshape(B, NP, 1)

    beta_row = beta.reshape(B, 1, N)
    sid_row = slice_id.reshape(B, 1, N)
    cp_row = is_cp.reshape(B, 1, N)

    out_shape = [jax.ShapeDtypeStruct((1, 1), jnp.float32)] * 4
    scalar_spec = pl.BlockSpec((1, 1), lambda b: (0, 0))
    outs = pl.pallas_call(
        _loss_kernel,
        grid=(B,),
        in_specs=[
            pl.BlockSpec((1, 1, N), lambda b: (b, 0, 0)),
            pl.BlockSpec((1, N, D), lambda b: (b, 0, 0)),
            pl.BlockSpec((1, 1, N), lambda b: (b, 0, 0)),
            pl.BlockSpec((1, 1, N), lambda b: (b, 0, 0)),
            pl.BlockSpec((1, NP, D), lambda b: (b, 0, 0)),
            pl.BlockSpec((1, 1, NP), lambda b: (b, 0, 0)),
            pl.BlockSpec((1, NP, 1), lambda b: (b, 0, 0)),
        ],
        out_specs=[scalar_spec] * 4,
        out_shape=out_shape,
        scratch_shapes=[pltpu.SMEM((8,), jnp.float32)],
    )(beta_row, embed, sid_row, cp_row, ce, csid_row, csid_col)
    return tuple(o.reshape(()) for o in outs)


# speedup vs baseline: 49.8069x; 1.0517x over previous
import jax
import jax.numpy as jnp
from jax import lax
from jax.experimental import pallas as pl
from jax.experimental.pallas import tpu as pltpu
from jax.experimental.pallas import tpu_sc as plsc

ATTR_W = 1.0
REP_W = 1.5
TAU = 0.7
SUPP_W = 0.5
CLAMP = 20.0

_B, _N, _D, _S = 4, 8192, 16, 128
_CAP = 1024
_NP = _CAP + 256
_TRP = 256
_CHUNK = _N // 8
_NV = _CHUNK // 16


def _sc_compact_kernel(cp_hbm, sid_hbm, emb_hbm, ce_hbm, csid_hbm,
                       cp_v, sid_v, lidx, lsid, cnt_v, tbl_sh, tbl_v,
                       srcb, dstb, rows, sidrows, sem):
    c = lax.axis_index("c")
    s = lax.axis_index("s")
    batch = c * 2 + s // 8
    rank = s % 8
    base = batch * _N + rank * _CHUNK
    lane = lax.iota(jnp.int32, 16)

    pltpu.async_copy(cp_hbm.at[pl.ds(base, _CHUNK)], cp_v, sem).wait()
    pltpu.async_copy(sid_hbm.at[pl.ds(base, _CHUNK)], sid_v, sem).wait()

    def body(i, wcount):
        o = i * 16
        cp16 = cp_v[pl.ds(o, 16)]
        sid16 = sid_v[pl.ds(o, 16)]
        m = cp16 == 1
        mi = jnp.where(m, 1, 0).astype(jnp.int32)
        cum = plsc.cumsum(mi)
        pos = wcount + cum - mi
        zeros = jnp.zeros((16,), jnp.int32)
        lidx[pl.ds(o, 16)] = zeros
        lsid[pl.ds(o, 16)] = zeros
        gidx = rank * _CHUNK + o + lane
        plsc.store_scatter(lidx, [pos], gidx, mask=m)
        plsc.store_scatter(lsid, [pos], sid16, mask=m)
        return wcount + jnp.sum(mi)

    wcount = lax.fori_loop(0, _NV, body, jnp.int32(0))

    cnt_v[...] = jnp.broadcast_to(wcount, (16,))
    pltpu.sync_copy(cnt_v, tbl_sh.at[s])
    plsc.subcore_barrier()
    pltpu.sync_copy(tbl_sh, tbl_v)
    g0 = (s // 8) * 8
    cnts = plsc.load_gather(tbl_v, [g0 + (lane & 7), jnp.zeros((16,), jnp.int32)])
    grp = jnp.where(lane < 8, cnts, 0)
    my_off = jnp.sum(jnp.where(lane < rank, grp, 0))

    dst_base = batch * _NP + my_off
    trash = batch * _NP + _CAP
    hi = batch * _NP + _NP - 1
    for ci in range(8):
        c0 = ci * 128

        @pl.when(wcount > c0)
        def _chunk():
            for v in range(8):
                o = c0 + v * 16
                li = lidx[pl.ds(o, 16)]
                ls = lsid[pl.ds(o, 16)]
                valid = (o + lane) < wcount
                srcb[pl.ds(v * 16, 16)] = batch * _N + li
                dst = jnp.where(valid, dst_base + o + lane, trash)
                dstb[pl.ds(v * 16, 16)] = jnp.minimum(dst, hi)
                plsc.store_scatter(
                    sidrows, [v * 16 + lane, jnp.zeros((16,), jnp.int32)], ls)
            pltpu.async_copy(emb_hbm.at[srcb], rows, sem).wait()
            pltpu.async_copy(rows, ce_hbm.at[dstb], sem).wait()
            pltpu.async_copy(sidrows, csid_hbm.at[dstb], sem).wait()


_sc_out_type = (jax.ShapeDtypeStruct((_B * _NP, _D), jnp.float32),
                jax.ShapeDtypeStruct((_B * _NP, 16), jnp.int32))
_sc_scratch = [
    pltpu.VMEM((_CHUNK,), jnp.int32),
    pltpu.VMEM((_CHUNK,), jnp.int32),
    pltpu.VMEM((_CHUNK,), jnp.int32),
    pltpu.VMEM((_CHUNK,), jnp.int32),
    pltpu.VMEM((16,), jnp.int32),
    pltpu.VMEM_SHARED((16, 16), jnp.int32),
    pltpu.VMEM((16, 16), jnp.int32),
    pltpu.VMEM((128,), jnp.int32),
    pltpu.VMEM((128,), jnp.int32),
    pltpu.VMEM((128, _D), jnp.float32),
    pltpu.VMEM((128, 16), jnp.int32),
    pltpu.SemaphoreType.DMA,
]


def _sc_compact(cp_flat, sid_flat, emb_flat):
    mesh = plsc.VectorSubcoreMesh(core_axis_name="c", subcore_axis_name="s")
    f = pl.kernel(_sc_compact_kernel, out_type=_sc_out_type, mesh=mesh,
                  scratch_types=_sc_scratch,
                  compiler_params=pltpu.CompilerParams(
                      needs_layout_passes=False,
                      use_tc_tiling_on_sc=False))
    return f(cp_flat, sid_flat, emb_flat)


def _stats_kernel(beta_ref, emb_ref, sid_row_ref, cp_row_ref, stat_ref):
    sid = sid_row_ref[0]
    cp = (cp_row_ref[0] == 1)
    bta = beta_ref[0]
    emb_raw = emb_ref[0]

    b2 = jnp.where(jnp.isfinite(bta), bta, 0.0)
    b2 = jnp.clip(b2, -CLAMP, CLAMP)
    embc = jnp.where(jnp.isfinite(emb_raw), emb_raw, 0.0)

    e = jnp.exp(b2 / TAU - CLAMP / TAU)
    cpf = cp.astype(jnp.float32)

    iota_s = lax.broadcasted_iota(jnp.int32, (_S, _N), 0)
    member = (sid == iota_s)
    memf = member.astype(jnp.float32)

    ones_row = jnp.ones((1, _N), jnp.float32)
    vals = jnp.concatenate([ones_row, cpf, e, e * cpf], axis=0)
    sums = lax.dot_general(memf, vals, (((1,), (1,)), ((), ())),
                           preferred_element_type=jnp.float32)
    counts = sums[:, 0:1]
    cp_counts = sums[:, 1:2]
    z_raw = sums[:, 2:3]
    cpz = sums[:, 3:4]
    z = jnp.where(counts > 0, z_raw, 1.0)

    p_cp = cpz / z
    slice_ce = -jnp.log(p_cp + 1e-9)
    noncp_sum = (z_raw - cpz) / z
    noncp_n = counts - cp_counts
    supp = jnp.where(noncp_n > 0, noncp_sum / jnp.maximum(noncp_n, 1.0), 0.0)
    sel = cp_counts == 1.0
    slice_count = jnp.sum(sel.astype(jnp.float32))
    beta_loss = jnp.sum(jnp.where(sel, slice_ce + SUPP_W * supp, 0.0))
    beta_loss = beta_loss / jnp.maximum(slice_count, 1.0)

    iota_n = lax.broadcasted_iota(jnp.int32, (_S, _N), 1)
    cpmem = member & cp
    min_idx = jnp.min(jnp.where(cpmem, iota_n, _N), axis=1, keepdims=True)
    onehot = (iota_n == min_idx).astype(jnp.float32)
    cp_vec = lax.dot_general(onehot, embc, (((1,), (0,)), ((), ())),
                             preferred_element_type=jnp.float32)

    emb2 = embc * embc
    nj = lax.dot_general(jnp.ones((1, _D), jnp.float32), emb2,
                         (((1,), (1,)), ((), ())),
                         preferred_element_type=jnp.float32)
    ns = jnp.sum(cp_vec * cp_vec, axis=1, keepdims=True)
    cross = lax.dot_general(cp_vec, embc, (((1,), (1,)), ((), ())),
                            preferred_element_type=jnp.float32)
    d2 = jnp.maximum(nj + ns - 2.0 * cross, 0.0)
    d2 = jnp.minimum(d2, 50.0)
    d2sum = jnp.sum(memf * d2, axis=1, keepdims=True)
    d2_mean = d2sum / jnp.maximum(counts, 1.0)
    att_sel = cp_counts > 0
    att_count = jnp.sum(att_sel.astype(jnp.float32))
    attraction = jnp.sum(jnp.where(att_sel, d2_mean, 0.0))
    attraction = jnp.where(att_count > 0,
                           ATTR_W * attraction / jnp.maximum(att_count, 1.0),
                           0.0)

    k_total = jnp.sum(cp_counts)
    rep_den = k_total * k_total - jnp.sum(cp_counts * cp_counts)
    cp_total = jnp.sum(cpf)
    vf = ((cp_total > 0) & (slice_count > 0)).astype(jnp.float32)

    one = jnp.ones((1, 1), jnp.float32)
    row = jnp.concatenate(
        [one * beta_loss, one * attraction, one * vf, one * k_total,
         one * rep_den, 0.0 * one, 0.0 * one, 0.0 * one], axis=1)
    stat_ref[...] = row.reshape(1, 1, 8)


def _rep_kernel(stat_ref, ce_ref, csid_row_ref, csid_col_ref, emb_ref,
                sid_row_ref, cp_row_ref, sid_col_ref, cp_col_ref,
                tot_ref, beta_out_ref, attr_out_ref, repl_out_ref, acc_ref):
    b = pl.program_id(0)
    nb = pl.num_programs(0)

    @pl.when(b == 0)
    def _init():
        for i in range(8):
            acc_ref[i] = 0.0

    k_total = stat_ref[b, 3]
    rep_den = stat_ref[b, 4]
    ki = k_total.astype(jnp.int32)
    ones_d = jnp.ones((1, _D), jnp.float32)

    ntiles = jnp.where(ki <= _CAP, (ki + _TRP - 1) // _TRP, 0)

    def _rep_tile(t, acc):
        tr = t // ntiles
        tc = t % ntiles
        r0 = tr * _TRP
        c0 = tc * _TRP
        cer_raw = ce_ref[0, pl.ds(r0, _TRP), :]
        cec_raw = ce_ref[0, pl.ds(c0, _TRP), :]
        ce_r = jnp.where(jnp.isfinite(cer_raw), cer_raw, 0.0)
        ce_c = jnp.where(jnp.isfinite(cec_raw), cec_raw, 0.0)
        gram = lax.dot_general(ce_r, ce_c, (((1,), (1,)), ((), ())),
                               preferred_element_type=jnp.float32)
        nr = jnp.sum(ce_r * ce_r, axis=1, keepdims=True)
        nc = lax.dot_general(ones_d, ce_c * ce_c, (((1,), (1,)), ((), ())),
                             preferred_element_type=jnp.float32)
        d2p = jnp.minimum(jnp.maximum(nr + nc - 2.0 * gram, 0.0), 50.0)
        sr = csid_col_ref[0, pl.ds(r0, _TRP), :]
        sc = csid_row_ref[0, :, pl.ds(c0, _TRP)]
        ir = lax.broadcasted_iota(jnp.int32, (_TRP, 1), 0) + r0
        ic = lax.broadcasted_iota(jnp.int32, (1, _TRP), 1) + c0
        pmask = (ir < ki) & (ic < ki) & (sr != sc)
        return acc + jnp.sum(jnp.where(pmask, jnp.exp(-d2p), 0.0))

    rep_fast = lax.fori_loop(0, ntiles * ntiles, _rep_tile, 0.0)

    nfull = jnp.where(ki > _CAP, _N // _TRP, 0)
    emb_raw = emb_ref[0]
    embc = jnp.where(jnp.isfinite(emb_raw), emb_raw, 0.0)
    emb2 = embc * embc
    nj = lax.dot_general(ones_d, emb2, (((1,), (1,)), ((), ())),
                         preferred_element_type=jnp.float32)
    sid = sid_row_ref[0]
    cp = (cp_row_ref[0] == 1)

    def _full_tile(t, acc):
        r0 = t * _TRP
        er_raw = emb_ref[0, pl.ds(r0, _TRP), :]
        emb_r = jnp.where(jnp.isfinite(er_raw), er_raw, 0.0)
        gram = lax.dot_general(emb_r, embc, (((1,), (1,)), ((), ())),
                               preferred_element_type=jnp.float32)
        nr = jnp.sum(emb_r * emb_r, axis=1, keepdims=True)
        d2p = jnp.minimum(jnp.maximum(nr + nj - 2.0 * gram, 0.0), 50.0)
        pmask = ((cp_col_ref[0, pl.ds(r0, _TRP), :] == 1) & cp
                 & (sid_col_ref[0, pl.ds(r0, _TRP), :] != sid))
        return acc + jnp.sum(jnp.where(pmask, jnp.exp(-d2p), 0.0))

    rep_num = lax.fori_loop(0, nfull, _full_tile, rep_fast)
    repulsion = jnp.where(rep_den > 0,
                          rep_num / jnp.maximum(rep_den, 1.0) * REP_W, 0.0)

    beta_loss = stat_ref[b, 0]
    attraction = stat_ref[b, 1]
    vf = stat_ref[b, 2]
    acc_ref[0] += vf * (beta_loss + attraction + repulsion)
    acc_ref[1] += vf * beta_loss
    acc_ref[2] += vf * attraction
    acc_ref[3] += vf * repulsion
    acc_ref[4] += vf

    @pl.when(b == nb - 1)
    def _fin():
        valid = acc_ref[4]
        denom = jnp.maximum(valid, 1.0)
        scale = jnp.where(valid > 0, 1.0 / denom, 0.0)
        one = jnp.ones((1, 1), jnp.float32)
        tot_ref[...] = one * (acc_ref[0] * scale)
        beta_out_ref[...] = one * (acc_ref[1] * scale)
        attr_out_ref[...] = one * (acc_ref[2] * scale)
        repl_out_ref[...] = one * (acc_ref[3] * scale)


def kernel(beta, embed, slice_id, is_cp):
    B, N, D = embed.shape
    NP = _NP
    ce2, csid2 = _sc_compact(is_cp.reshape(B * N),
                             slice_id.reshape(B * N),
                             embed.reshape(B * N, D))
    ce = ce2.reshape(B, NP, D)
    csid = csid2[:, 0]
    csid_row = csid.reshape(B, 1, NP)
    csid_col = csid.reshape(B, NP, 1)

    beta_row = beta.reshape(B, 1, N)
    sid_row = slice_id.reshape(B, 1, N)
    cp_row = is_cp.reshape(B, 1, N)
    sid_col = slice_id.reshape(B, N, 1)
    cp_col = is_cp.reshape(B, N, 1)

    stats = pl.pallas_call(
        _stats_kernel,
        grid=(B,),
        in_specs=[
            pl.BlockSpec((1, 1, N), lambda b: (b, 0, 0)),
            pl.BlockSpec((1, N, D), lambda b: (b, 0, 0)),
            pl.BlockSpec((1, 1, N), lambda b: (b, 0, 0)),
            pl.BlockSpec((1, 1, N), lambda b: (b, 0, 0)),
        ],
        out_specs=pl.BlockSpec((1, 1, 8), lambda b: (b, 0, 0)),
        out_shape=jax.ShapeDtypeStruct((B, 1, 8), jnp.float32),
    )(beta_row, embed, sid_row, cp_row)

    out_shape = [jax.ShapeDtypeStruct((1, 1), jnp.float32)] * 4
    scalar_spec = pl.BlockSpec((1, 1), lambda b: (0, 0))
    outs = pl.pallas_call(
        _rep_kernel,
        grid=(B,),
        in_specs=[
            pl.BlockSpec(memory_space=pltpu.SMEM),
            pl.BlockSpec((1, NP, D), lambda b: (b, 0, 0)),
            pl.BlockSpec((1, 1, NP), lambda b: (b, 0, 0)),
            pl.BlockSpec((1, NP, 1), lambda b: (b, 0, 0)),
            pl.BlockSpec((1, N, D), lambda b: (b, 0, 0)),
            pl.BlockSpec((1, 1, N), lambda b: (b, 0, 0)),
            pl.BlockSpec((1, 1, N), lambda b: (b, 0, 0)),
            pl.BlockSpec((1, N, 1), lambda b: (b, 0, 0)),
            pl.BlockSpec((1, N, 1), lambda b: (b, 0, 0)),
        ],
        out_specs=[scalar_spec] * 4,
        out_shape=out_shape,
        scratch_shapes=[pltpu.SMEM((8,), jnp.float32)],
    )(stats.reshape(B, 8), ce, csid_row, csid_col, embed,
      sid_row, cp_row, sid_col, cp_col)
    return tuple(o.reshape(()) for o in outs)


# speedup vs baseline: 72.7712x; 1.4611x over previous
import jax
import jax.numpy as jnp
from jax import lax
from jax.experimental import pallas as pl
from jax.experimental.pallas import tpu as pltpu
from jax.experimental.pallas import tpu_sc as plsc

ATTR_W = 1.0
REP_W = 1.5
TAU = 0.7
SUPP_W = 0.5
CLAMP = 20.0

_B, _N, _D, _S = 4, 8192, 16, 128
_CAP = 1024
_NP = _CAP + 256
_TRP = 256
_CHUNK = _N // 8
_NV = _CHUNK // 16


def _sc_compact_kernel(cp_hbm, sid_hbm, emb_hbm, ce_hbm, csid_hbm,
                       cp_v, sid_v, lidx, lsid, cnt_v, tbl_sh, tbl_v,
                       srcb, dstb, rows, sidrows, sem):
    c = lax.axis_index("c")
    s = lax.axis_index("s")
    batch = c * 2 + s // 8
    rank = s % 8
    base = batch * _N + rank * _CHUNK
    lane = lax.iota(jnp.int32, 16)

    pltpu.async_copy(cp_hbm.at[pl.ds(base, _CHUNK)], cp_v, sem).wait()
    pltpu.async_copy(sid_hbm.at[pl.ds(base, _CHUNK)], sid_v, sem).wait()

    def body(i, wcount):
        o = i * 16
        cp16 = cp_v[pl.ds(o, 16)]
        sid16 = sid_v[pl.ds(o, 16)]
        m = cp16 == 1
        mi = jnp.where(m, 1, 0).astype(jnp.int32)
        cum = plsc.cumsum(mi)
        pos = wcount + cum - mi
        zeros = jnp.zeros((16,), jnp.int32)
        lidx[pl.ds(o, 16)] = zeros
        lsid[pl.ds(o, 16)] = zeros
        gidx = rank * _CHUNK + o + lane
        plsc.store_scatter(lidx, [pos], gidx, mask=m)
        plsc.store_scatter(lsid, [pos], sid16, mask=m)
        return wcount + jnp.sum(mi)

    wcount = lax.fori_loop(0, _NV, body, jnp.int32(0))

    cnt_v[...] = jnp.broadcast_to(wcount, (16,))
    pltpu.sync_copy(cnt_v, tbl_sh.at[s])
    plsc.subcore_barrier()
    pltpu.sync_copy(tbl_sh, tbl_v)
    g0 = (s // 8) * 8
    cnts = plsc.load_gather(tbl_v, [g0 + (lane & 7), jnp.zeros((16,), jnp.int32)])
    grp = jnp.where(lane < 8, cnts, 0)
    my_off = jnp.sum(jnp.where(lane < rank, grp, 0))

    dst_base = batch * _NP + my_off
    trash = batch * _NP + _CAP
    hi = batch * _NP + _NP - 1
    for ci in range(8):
        c0 = ci * 128

        @pl.when(wcount > c0)
        def _chunk():
            for v in range(8):
                o = c0 + v * 16
                li = lidx[pl.ds(o, 16)]
                ls = lsid[pl.ds(o, 16)]
                valid = (o + lane) < wcount
                srcb[pl.ds(v * 16, 16)] = batch * _N + li
                dst = jnp.where(valid, dst_base + o + lane, trash)
                dstb[pl.ds(v * 16, 16)] = jnp.minimum(dst, hi)
                plsc.store_scatter(
                    sidrows, [v * 16 + lane, jnp.zeros((16,), jnp.int32)], ls)
            pltpu.async_copy(emb_hbm.at[srcb], rows, sem).wait()
            pltpu.async_copy(rows, ce_hbm.at[dstb], sem).wait()
            pltpu.async_copy(sidrows, csid_hbm.at[dstb], sem).wait()


_sc_out_type = (jax.ShapeDtypeStruct((_B * _NP, _D), jnp.float32),
                jax.ShapeDtypeStruct((_B * _NP, 16), jnp.int32))
_sc_scratch = [
    pltpu.VMEM((_CHUNK,), jnp.int32),
    pltpu.VMEM((_CHUNK,), jnp.int32),
    pltpu.VMEM((_CHUNK,), jnp.int32),
    pltpu.VMEM((_CHUNK,), jnp.int32),
    pltpu.VMEM((16,), jnp.int32),
    pltpu.VMEM_SHARED((16, 16), jnp.int32),
    pltpu.VMEM((16, 16), jnp.int32),
    pltpu.VMEM((128,), jnp.int32),
    pltpu.VMEM((128,), jnp.int32),
    pltpu.VMEM((128, _D), jnp.float32),
    pltpu.VMEM((128, 16), jnp.int32),
    pltpu.SemaphoreType.DMA,
]


def _sc_compact(cp_flat, sid_flat, emb_flat):
    mesh = plsc.VectorSubcoreMesh(core_axis_name="c", subcore_axis_name="s")
    f = pl.kernel(_sc_compact_kernel, out_type=_sc_out_type, mesh=mesh,
                  scratch_types=_sc_scratch,
                  compiler_params=pltpu.CompilerParams(
                      needs_layout_passes=False,
                      use_tc_tiling_on_sc=False))
    return f(cp_flat, sid_flat, emb_flat)


def _stats_kernel(beta_ref, emb_ref, sid_row_ref, cp_row_ref, stat_ref):
    sid = sid_row_ref[0]
    cp = (cp_row_ref[0] == 1)
    bta = beta_ref[0]
    emb_raw = emb_ref[0]

    b2 = jnp.where(jnp.isfinite(bta), bta, 0.0)
    b2 = jnp.clip(b2, -CLAMP, CLAMP)
    embc = jnp.where(jnp.isfinite(emb_raw), emb_raw, 0.0)

    e = jnp.exp(b2 / TAU - CLAMP / TAU)
    cpf = cp.astype(jnp.float32)

    iota_s = lax.broadcasted_iota(jnp.int32, (_S, _N), 0)
    member = (sid == iota_s)
    memf = member.astype(jnp.float32)

    ones_row = jnp.ones((1, _N), jnp.float32)
    vals = jnp.concatenate([ones_row, cpf, e, e * cpf], axis=0)
    sums = lax.dot_general(memf, vals, (((1,), (1,)), ((), ())),
                           preferred_element_type=jnp.float32)
    counts = sums[:, 0:1]
    cp_counts = sums[:, 1:2]
    z_raw = sums[:, 2:3]
    cpz = sums[:, 3:4]
    z = jnp.where(counts > 0, z_raw, 1.0)

    p_cp = cpz / z
    slice_ce = -jnp.log(p_cp + 1e-9)
    noncp_sum = (z_raw - cpz) / z
    noncp_n = counts - cp_counts
    supp = jnp.where(noncp_n > 0, noncp_sum / jnp.maximum(noncp_n, 1.0), 0.0)
    sel = cp_counts == 1.0
    slice_count = jnp.sum(sel.astype(jnp.float32))
    beta_loss = jnp.sum(jnp.where(sel, slice_ce + SUPP_W * supp, 0.0))
    beta_loss = beta_loss / jnp.maximum(slice_count, 1.0)

    iota_n = lax.broadcasted_iota(jnp.int32, (_S, _N), 1)
    cpmem = member & cp
    min_idx = jnp.min(jnp.where(cpmem, iota_n, _N), axis=1, keepdims=True)
    onehot = (iota_n == min_idx).astype(jnp.float32)
    cp_vec = lax.dot_general(onehot, embc, (((1,), (0,)), ((), ())),
                             preferred_element_type=jnp.float32)

    emb2 = embc * embc
    nj = lax.dot_general(jnp.ones((1, _D), jnp.float32), emb2,
                         (((1,), (1,)), ((), ())),
                         preferred_element_type=jnp.float32)
    ns = jnp.sum(cp_vec * cp_vec, axis=1, keepdims=True)
    cross = lax.dot_general(cp_vec, embc, (((1,), (1,)), ((), ())),
                            preferred_element_type=jnp.float32)
    d2 = jnp.maximum(nj + ns - 2.0 * cross, 0.0)
    d2 = jnp.minimum(d2, 50.0)
    d2sum = jnp.sum(memf * d2, axis=1, keepdims=True)
    d2_mean = d2sum / jnp.maximum(counts, 1.0)
    att_sel = cp_counts > 0
    att_count = jnp.sum(att_sel.astype(jnp.float32))
    attraction = jnp.sum(jnp.where(att_sel, d2_mean, 0.0))
    attraction = jnp.where(att_count > 0,
                           ATTR_W * attraction / jnp.maximum(att_count, 1.0),
                           0.0)

    k_total = jnp.sum(cp_counts)
    rep_den = k_total * k_total - jnp.sum(cp_counts * cp_counts)
    cp_total = jnp.sum(cpf)
    vf = ((cp_total > 0) & (slice_count > 0)).astype(jnp.float32)

    one = jnp.ones((1, 1), jnp.float32)
    row = jnp.concatenate(
        [one * beta_loss, one * attraction, one * vf, one * k_total,
         one * rep_den, 0.0 * one, 0.0 * one, 0.0 * one], axis=1)
    stat_ref[...] = row.reshape(1, 1, 8)


def _fallback_kernel(emb_ref, sid_row_ref, cp_row_ref, sid_col_ref,
                     cp_col_ref, out_ref):
    emb_raw = emb_ref[0]
    embc = jnp.where(jnp.isfinite(emb_raw), emb_raw, 0.0)
    ones_d = jnp.ones((1, _D), jnp.float32)
    nj = lax.dot_general(ones_d, embc * embc, (((1,), (1,)), ((), ())),
                         preferred_element_type=jnp.float32)
    sid = sid_row_ref[0]
    cp = (cp_row_ref[0] == 1)

    def _full_tile(t, acc):
        r0 = t * _TRP
        er_raw = emb_ref[0, pl.ds(r0, _TRP), :]
        emb_r = jnp.where(jnp.isfinite(er_raw), er_raw, 0.0)
        gram = lax.dot_general(emb_r, embc, (((1,), (1,)), ((), ())),
                               preferred_element_type=jnp.float32)
        nr = jnp.sum(emb_r * emb_r, axis=1, keepdims=True)
        d2p = jnp.minimum(jnp.maximum(nr + nj - 2.0 * gram, 0.0), 50.0)
        pmask = ((cp_col_ref[0, pl.ds(r0, _TRP), :] == 1) & cp
                 & (sid_col_ref[0, pl.ds(r0, _TRP), :] != sid))
        return acc + jnp.sum(jnp.where(pmask, jnp.exp(-d2p), 0.0))

    rep_num = lax.fori_loop(0, _N // _TRP, _full_tile, 0.0)
    out_ref[...] = jnp.ones((1, 1, 1), jnp.float32) * rep_num


def _fallback(embed, slice_id, is_cp):
    B, N, D = embed.shape
    return pl.pallas_call(
        _fallback_kernel,
        grid=(B,),
        in_specs=[
            pl.BlockSpec((1, N, D), lambda b: (b, 0, 0)),
            pl.BlockSpec((1, 1, N), lambda b: (b, 0, 0)),
            pl.BlockSpec((1, 1, N), lambda b: (b, 0, 0)),
            pl.BlockSpec((1, N, 1), lambda b: (b, 0, 0)),
            pl.BlockSpec((1, N, 1), lambda b: (b, 0, 0)),
        ],
        out_specs=pl.BlockSpec((1, 1, 1), lambda b: (b, 0, 0)),
        out_shape=jax.ShapeDtypeStruct((B, 1, 1), jnp.float32),
    )(embed, slice_id.reshape(B, 1, N), is_cp.reshape(B, 1, N),
      slice_id.reshape(B, N, 1), is_cp.reshape(B, N, 1)).reshape(B)


def _rep_kernel(stat_ref, fb_ref, ce_ref, csid_row_ref, csid_col_ref,
                tot_ref, beta_out_ref, attr_out_ref, repl_out_ref, acc_ref):
    b = pl.program_id(0)
    nb = pl.num_programs(0)

    @pl.when(b == 0)
    def _init():
        for i in range(8):
            acc_ref[i] = 0.0

    k_total = stat_ref[b, 3]
    rep_den = stat_ref[b, 4]
    ki = k_total.astype(jnp.int32)
    ones_d = jnp.ones((1, _D), jnp.float32)

    ntiles = jnp.where(ki <= _CAP, (ki + _TRP - 1) // _TRP, 0)

    def _rep_tile(t, acc):
        tr = t // ntiles
        tc = t % ntiles
        r0 = tr * _TRP
        c0 = tc * _TRP
        cer_raw = ce_ref[0, pl.ds(r0, _TRP), :]
        cec_raw = ce_ref[0, pl.ds(c0, _TRP), :]
        ce_r = jnp.where(jnp.isfinite(cer_raw), cer_raw, 0.0)
        ce_c = jnp.where(jnp.isfinite(cec_raw), cec_raw, 0.0)
        gram = lax.dot_general(ce_r, ce_c, (((1,), (1,)), ((), ())),
                               preferred_element_type=jnp.float32)
        nr = jnp.sum(ce_r * ce_r, axis=1, keepdims=True)
        nc = lax.dot_general(ones_d, ce_c * ce_c, (((1,), (1,)), ((), ())),
                             preferred_element_type=jnp.float32)
        d2p = jnp.minimum(jnp.maximum(nr + nc - 2.0 * gram, 0.0), 50.0)
        sr = csid_col_ref[0, pl.ds(r0, _TRP), :]
        sc = csid_row_ref[0, :, pl.ds(c0, _TRP)]
        ir = lax.broadcasted_iota(jnp.int32, (_TRP, 1), 0) + r0
        ic = lax.broadcasted_iota(jnp.int32, (1, _TRP), 1) + c0
        pmask = (ir < ki) & (ic < ki) & (sr != sc)
        return acc + jnp.sum(jnp.where(pmask, jnp.exp(-d2p), 0.0))

    rep_fast = lax.fori_loop(0, ntiles * ntiles, _rep_tile, 0.0)

    rep_num = jnp.where(ki <= _CAP, rep_fast, fb_ref[b])
    repulsion = jnp.where(rep_den > 0,
                          rep_num / jnp.maximum(rep_den, 1.0) * REP_W, 0.0)

    beta_loss = stat_ref[b, 0]
    attraction = stat_ref[b, 1]
    vf = stat_ref[b, 2]
    acc_ref[0] += vf * (beta_loss + attraction + repulsion)
    acc_ref[1] += vf * beta_loss
    acc_ref[2] += vf * attraction
    acc_ref[3] += vf * repulsion
    acc_ref[4] += vf

    @pl.when(b == nb - 1)
    def _fin():
        valid = acc_ref[4]
        denom = jnp.maximum(valid, 1.0)
        scale = jnp.where(valid > 0, 1.0 / denom, 0.0)
        one = jnp.ones((1, 1), jnp.float32)
        tot_ref[...] = one * (acc_ref[0] * scale)
        beta_out_ref[...] = one * (acc_ref[1] * scale)
        attr_out_ref[...] = one * (acc_ref[2] * scale)
        repl_out_ref[...] = one * (acc_ref[3] * scale)


def kernel(beta, embed, slice_id, is_cp):
    B, N, D = embed.shape
    NP = _NP
    ce2, csid2 = _sc_compact(is_cp.reshape(B * N),
                             slice_id.reshape(B * N),
                             embed.reshape(B * N, D))
    ce = ce2.reshape(B, NP, D)
    csid = csid2[:, 0]
    csid_row = csid.reshape(B, 1, NP)
    csid_col = csid.reshape(B, NP, 1)

    beta_row = beta.reshape(B, 1, N)
    sid_row = slice_id.reshape(B, 1, N)
    cp_row = is_cp.reshape(B, 1, N)

    stats = pl.pallas_call(
        _stats_kernel,
        grid=(B,),
        in_specs=[
            pl.BlockSpec((1, 1, N), lambda b: (b, 0, 0)),
            pl.BlockSpec((1, N, D), lambda b: (b, 0, 0)),
            pl.BlockSpec((1, 1, N), lambda b: (b, 0, 0)),
            pl.BlockSpec((1, 1, N), lambda b: (b, 0, 0)),
        ],
        out_specs=pl.BlockSpec((1, 1, 8), lambda b: (b, 0, 0)),
        out_shape=jax.ShapeDtypeStruct((B, 1, 8), jnp.float32),
    )(beta_row, embed, sid_row, cp_row)

    ktot = stats[:, 0, 3]
    fb = lax.cond(jnp.any(ktot > float(_CAP)),
                  lambda: _fallback(embed, slice_id, is_cp),
                  lambda: jnp.zeros((B,), jnp.float32))

    out_shape = [jax.ShapeDtypeStruct((1, 1), jnp.float32)] * 4
    scalar_spec = pl.BlockSpec((1, 1), lambda b: (0, 0))
    outs = pl.pallas_call(
        _rep_kernel,
        grid=(B,),
        in_specs=[
            pl.BlockSpec(memory_space=pltpu.SMEM),
            pl.BlockSpec(memory_space=pltpu.SMEM),
            pl.BlockSpec((1, NP, D), lambda b: (b, 0, 0)),
            pl.BlockSpec((1, 1, NP), lambda b: (b, 0, 0)),
            pl.BlockSpec((1, NP, 1), lambda b: (b, 0, 0)),
        ],
        out_specs=[scalar_spec] * 4,
        out_shape=out_shape,
        scratch_shapes=[pltpu.SMEM((8,), jnp.float32)],
    )(stats.reshape(B, 8), fb, ce, csid_row, csid_col)
    return tuple(o.reshape(()) for o in outs)
